# baseline (speedup 1.0000x reference)
import functools

import jax
import jax.numpy as jnp
from jax import lax
from jax.experimental import pallas as pl
from jax.experimental.pallas import tpu as pltpu
from jax.experimental.pallas import tpu_sc as plsc

CHUNK = 128
NBUF = 4

_NUM_CORES = 2
_NUM_SUBCORES = 16
_NW = _NUM_CORES * _NUM_SUBCORES


@functools.cache
def _build(n_rows: int, dim: int):
    assert n_rows % (_NW * CHUNK) == 0
    nchunks_w = n_rows // (_NW * CHUNK)
    assert nchunks_w % NBUF == 0

    scratch = [pltpu.VMEM((nchunks_w, CHUNK), jnp.int32)]
    scratch += [pltpu.VMEM((CHUNK, dim), jnp.float32) for _ in range(NBUF)]
    scratch += [pltpu.SemaphoreType.DMA for _ in range(NBUF)]

    @functools.partial(
        pl.kernel,
        mesh=plsc.VectorSubcoreMesh(core_axis_name="c", subcore_axis_name="s"),
        out_type=jax.ShapeDtypeStruct((n_rows, dim), jnp.float32),
        scratch_types=scratch,
        compiler_params=pltpu.CompilerParams(use_tc_tiling_on_sc=False),
    )
    def emb(idx_hbm, table_hbm, out_hbm, idx_v, *rest):
        bufs = rest[:NBUF]
        sems = rest[NBUF:]
        wid = lax.axis_index("s") * _NUM_CORES + lax.axis_index("c")
        chunk0 = wid * nchunks_w

        pltpu.sync_copy(idx_hbm.at[pl.ds(chunk0, nchunks_w)], idx_v)

        for b in range(NBUF):
            pltpu.async_copy(table_hbm.at[idx_v.at[b]], bufs[b], sems[b])

        def round_(i, carry):
            for b in range(NBUF):
                j = i * NBUF + b
                pltpu.make_async_copy(
                    table_hbm.at[idx_v.at[j]], bufs[b], sems[b]).wait()
                pltpu.sync_copy(
                    bufs[b], out_hbm.at[pl.ds((chunk0 + j) * CHUNK, CHUNK)])
                nxt = j + NBUF

                @pl.when(nxt < nchunks_w)
                def _():
                    pltpu.async_copy(
                        table_hbm.at[idx_v.at[nxt]], bufs[b], sems[b])
            return carry

        lax.fori_loop(0, nchunks_w // NBUF, round_, 0)

    return emb


def kernel(token_ids, embedding):
    bsz, fields = token_ids.shape
    _, dim = embedding.shape
    n_rows = bsz * fields
    idx = token_ids.reshape(n_rows // CHUNK, CHUNK).astype(jnp.int32)
    out = _build(n_rows, dim)(idx, embedding)
    return out.reshape(bsz, fields, dim)
